# baseline (speedup 1.0000x reference)
import functools

import jax
import jax.numpy as jnp
from jax import lax
from jax.experimental import pallas as pl
from jax.experimental.pallas import tpu as pltpu
from jax.experimental.pallas import tpu_sc as plsc

_NUM_CORES = 2
_NUM_SUBCORES = 16
_NUM_WORKERS = _NUM_CORES * _NUM_SUBCORES

_TG = 5
_L = 16


@functools.lru_cache(maxsize=None)
def _make_gather(batch: int, hist: int, dim: int):
    assert dim == 32
    assert batch % (128 * _NUM_WORKERS) == 0
    assert hist % _TG == 0
    n_bt = batch // (128 * _NUM_WORKERS)
    n_g = hist // _TG
    n_groups = n_bt * n_g
    assert n_groups % 2 == 0 and n_bt >= 2
    rpg = 128 * _TG
    n_row2 = _TG * (dim // 8)

    mesh = plsc.VectorSubcoreMesh(core_axis_name="c", subcore_axis_name="s")

    @functools.partial(
        pl.kernel,
        mesh=mesh,
        compiler_params=pltpu.CompilerParams(
            use_tc_tiling_on_sc=False, needs_layout_passes=False
        ),
        out_type=jax.ShapeDtypeStruct(
            (hist, dim // 8, batch // 128, 8, 128), jnp.float32
        ),
        scratch_types=[
            pltpu.VMEM((2, 128 * hist), jnp.int32),
            pltpu.VMEM((2, rpg), jnp.int32),
            pltpu.VMEM((2, rpg, dim), jnp.float32),
            pltpu.VMEM((2, n_row2 * 8, 128), jnp.float32),
            pltpu.SemaphoreType.DMA,
            pltpu.SemaphoreType.DMA,
            pltpu.SemaphoreType.DMA,
            pltpu.SemaphoreType.DMA,
            pltpu.SemaphoreType.DMA,
        ],
    )
    def gather_kernel(
        idx_hbm, table_hbm, out_hbm, idx_v, idxg_v, rows_v, tiles_v, si, sg0, sg1, so0, so1
    ):
        wid = lax.axis_index("s") * _NUM_CORES + lax.axis_index("c")
        sg = (sg0, sg1)
        so = (so0, so1)

        iota = lax.broadcasted_iota(jnp.int32, (_L,), 0)
        iota_h = iota * hist
        j0 = iota
        j1 = iota + _L
        jt0 = jnp.right_shift(j0, 3)
        js0 = jnp.bitwise_and(j0, 7)
        jt1 = jnp.right_shift(j1, 3)
        js1 = jnp.bitwise_and(j1, 7)

        def idx_stage(bt_local, slot):
            r0 = (wid * n_bt + bt_local) * 128 * hist
            return pltpu.make_async_copy(
                idx_hbm.at[pl.ds(r0, 128 * hist)], idx_v.at[slot], si
            )

        def build_idxg(g, slot, ibt):
            t0 = (g % n_g) * _TG
            for tt in range(_TG):
                for lg in range(8):
                    pos = iota_h + ((lg * _L) * hist + t0 + tt)
                    vals = plsc.load_gather(idx_v.at[ibt], [pos])
                    idxg_v[slot, pl.ds(tt * 128 + lg * _L, _L)] = vals

        def gather(slot):
            return pltpu.make_async_copy(
                table_hbm.at[idxg_v.at[slot]], rows_v.at[slot], sg[slot]
            )

        def transpose(slot):
            for tt in range(_TG):
                rj0 = j0 + tt * 32
                rj1 = j1 + tt * 32

                def tp_body(bl, tt=tt, rj0=rj0, rj1=rj1):
                    r = tt * 128 + bl
                    bl_vec = jnp.full((_L,), 0, jnp.int32) + bl
                    v0 = rows_v[slot, r, pl.ds(0, _L)]
                    v1 = rows_v[slot, r, pl.ds(_L, _L)]
                    plsc.store_scatter(tiles_v.at[slot], [rj0, bl_vec], v0)
                    plsc.store_scatter(tiles_v.at[slot], [rj1, bl_vec], v1)

                plsc.parallel_loop(0, 128, 1, unroll=16)(tp_body)

        def outs_start(g, slot):
            t0 = (g % n_g) * _TG
            bt = wid * n_bt + g // n_g
            for tt in range(_TG):
                for jt in range(dim // 8):
                    pltpu.async_copy(
                        tiles_v.at[slot, pl.ds((tt * 4 + jt) * 8, 8)],
                        out_hbm.at[t0 + tt, jt, bt],
                        so[slot],
                    )

        def outs_wait(g, slot):
            t0 = (g % n_g) * _TG
            bt = wid * n_bt + g // n_g
            for tt in range(_TG):
                for jt in range(dim // 8):
                    pltpu.make_async_copy(
                        tiles_v.at[slot, pl.ds((tt * 4 + jt) * 8, 8)],
                        out_hbm.at[t0 + tt, jt, bt],
                        so[slot],
                    ).wait()

        idx_stage(0, 0).start()
        idx_stage(0, 0).wait()
        build_idxg(0, 0, 0)
        gather(0).start()

        def step(g, s):
            o = 1 - s
            gather(s).wait()
            g1 = g + 1
            ibt1 = (g1 // n_g) % 2

            @pl.when(jnp.logical_and(g1 % n_g == 0, g1 < n_groups))
            def _():
                idx_stage(g1 // n_g, ibt1).wait()

            @pl.when(g1 < n_groups)
            def _():
                build_idxg(g1, o, ibt1)
                gather(o).start()

            @pl.when(g >= 2)
            def _():
                outs_wait(g - 2, s)

            transpose(s)
            outs_start(g, s)

            @pl.when(jnp.logical_and(g % n_g == 0, g < n_groups - n_g))
            def _():
                idx_stage(g // n_g + 1, (g // n_g + 1) % 2).start()

        def body(k, carry):
            step(2 * k, 0)
            step(2 * k + 1, 1)
            return carry

        lax.fori_loop(0, n_groups // 2, body, 0)
        outs_wait(n_groups - 2, 0)
        outs_wait(n_groups - 1, 1)

    return gather_kernel


def kernel(x, weight):
    batch, hist = x.shape
    dim = weight.shape[1]
    flat_idx = x.reshape(-1).astype(jnp.int32)
    out5 = _make_gather(batch, hist, dim)(flat_idx, weight)
    return lax.reshape(out5, (batch, hist, dim), dimensions=(2, 4, 0, 1, 3))


# speedup vs baseline: 1.0502x; 1.0502x over previous
import functools

import jax
import jax.numpy as jnp
from jax import lax
from jax.experimental import pallas as pl
from jax.experimental.pallas import tpu as pltpu
from jax.experimental.pallas import tpu_sc as plsc

_NUM_CORES = 2
_NUM_SUBCORES = 16
_NUM_WORKERS = _NUM_CORES * _NUM_SUBCORES

_TG = 5
_L = 16


@functools.lru_cache(maxsize=None)
def _make_gather(batch: int, hist: int, dim: int):
    assert dim == 32
    assert batch % (128 * _NUM_WORKERS) == 0
    assert hist % _TG == 0
    n_bt = batch // (128 * _NUM_WORKERS)
    n_g = hist // _TG
    n_groups = n_bt * n_g
    assert n_groups % 2 == 0 and n_bt >= 2
    rpg = 128 * _TG
    n_row2 = _TG * (dim // 8)

    mesh = plsc.VectorSubcoreMesh(core_axis_name="c", subcore_axis_name="s")

    @functools.partial(
        pl.kernel,
        mesh=mesh,
        compiler_params=pltpu.CompilerParams(
            use_tc_tiling_on_sc=False, needs_layout_passes=False
        ),
        out_type=jax.ShapeDtypeStruct(
            (hist, dim // 8, batch // 128, 8, 128), jnp.float32
        ),
        scratch_types=[
            pltpu.VMEM((2, 128 * hist), jnp.int32),
            pltpu.VMEM((2, rpg), jnp.int32),
            pltpu.VMEM((2, rpg, dim), jnp.float32),
            pltpu.VMEM((2, n_row2 * 8, 128), jnp.float32),
            pltpu.SemaphoreType.DMA,
            pltpu.SemaphoreType.DMA,
            pltpu.SemaphoreType.DMA,
            pltpu.SemaphoreType.DMA,
            pltpu.SemaphoreType.DMA,
        ],
    )
    def gather_kernel(
        idx_hbm, table_hbm, out_hbm, idx_v, idxg_v, rows_v, tiles_v, si, sg0, sg1, so0, so1
    ):
        wid = lax.axis_index("s") * _NUM_CORES + lax.axis_index("c")
        sg = (sg0, sg1)
        so = (so0, so1)

        iota = lax.broadcasted_iota(jnp.int32, (_L,), 0)
        iota_h = iota * hist
        j0 = iota
        j1 = iota + _L

        def idx_stage(bt_local, slot):
            r0 = (wid * n_bt + bt_local) * 128 * hist
            return pltpu.make_async_copy(
                idx_hbm.at[pl.ds(r0, 128 * hist)], idx_v.at[slot], si
            )

        def build_idxg(g, slot, ibt):
            t0 = (g % n_g) * _TG
            for tt in range(_TG):
                for lg in range(8):
                    pos = iota_h + ((lg * _L) * hist + t0 + tt)
                    vals = plsc.load_gather(idx_v.at[ibt], [pos])
                    idxg_v[slot, pl.ds(tt * 128 + lg * _L, _L)] = vals

        def gather(slot):
            return pltpu.make_async_copy(
                table_hbm.at[idxg_v.at[slot]], rows_v.at[slot], sg[slot]
            )

        def transpose(slot):
            for tt in range(_TG):
                rj0 = j0 + tt * 32
                rj1 = j1 + tt * 32

                def tp_body(bl, tt=tt, rj0=rj0, rj1=rj1):
                    r = tt * 128 + bl
                    bl_vec = jnp.full((_L,), 0, jnp.int32) + bl
                    v0 = rows_v[slot, r, pl.ds(0, _L)]
                    v1 = rows_v[slot, r, pl.ds(_L, _L)]
                    plsc.store_scatter(tiles_v.at[slot], [rj0, bl_vec], v0)
                    plsc.store_scatter(tiles_v.at[slot], [rj1, bl_vec], v1)

                plsc.parallel_loop(0, 128, 1, unroll=8)(tp_body)

        def outs_start(g, slot):
            t0 = (g % n_g) * _TG
            bt = wid * n_bt + g // n_g
            for tt in range(_TG):
                for jt in range(dim // 8):
                    pltpu.async_copy(
                        tiles_v.at[slot, pl.ds((tt * 4 + jt) * 8, 8)],
                        out_hbm.at[t0 + tt, jt, bt],
                        so[slot],
                    )

        def outs_wait(g, slot):
            t0 = (g % n_g) * _TG
            bt = wid * n_bt + g // n_g
            for tt in range(_TG):
                for jt in range(dim // 8):
                    pltpu.make_async_copy(
                        tiles_v.at[slot, pl.ds((tt * 4 + jt) * 8, 8)],
                        out_hbm.at[t0 + tt, jt, bt],
                        so[slot],
                    ).wait()

        idx_stage(0, 0).start()
        idx_stage(0, 0).wait()
        build_idxg(0, 0, 0)
        gather(0).start()

        def step(g, s):
            o = 1 - s
            gather(s).wait()
            g1 = g + 1
            ibt1 = (g1 // n_g) % 2

            @pl.when(jnp.logical_and(g1 % n_g == 0, g1 < n_groups))
            def _():
                idx_stage(g1 // n_g, ibt1).wait()

            @pl.when(g1 < n_groups)
            def _():
                build_idxg(g1, o, ibt1)
                gather(o).start()

            @pl.when(g >= 2)
            def _():
                outs_wait(g - 2, s)

            transpose(s)
            outs_start(g, s)

            @pl.when(jnp.logical_and(g % n_g == 0, g < n_groups - n_g))
            def _():
                idx_stage(g // n_g + 1, (g // n_g + 1) % 2).start()

        def body(k, carry):
            step(2 * k, 0)
            step(2 * k + 1, 1)
            return carry

        lax.fori_loop(0, n_groups // 2, body, 0)
        outs_wait(n_groups - 2, 0)
        outs_wait(n_groups - 1, 1)

    return gather_kernel


def kernel(x, weight):
    batch, hist = x.shape
    dim = weight.shape[1]
    flat_idx = x.reshape(-1).astype(jnp.int32)
    out5 = _make_gather(batch, hist, dim)(flat_idx, weight)
    return lax.reshape(out5, (batch, hist, dim), dimensions=(2, 4, 0, 1, 3))
